# baseline (speedup 1.0000x reference)
import functools

import jax
import jax.numpy as jnp
from jax import lax
from jax.experimental import pallas as pl
from jax.experimental.pallas import tpu as pltpu
from jax.experimental.pallas import tpu_sc as plsc

_F32 = jnp.float32


def _mm_body(x_ref, w_ref, o_ref, *, relu):
    acc = jnp.dot(x_ref[...], w_ref[...], preferred_element_type=_F32)
    if relu:
        acc = jnp.maximum(acc, 0.0)
    o_ref[...] = acc


def _mm(x, w, relu=False):
    M, K = x.shape
    N = w.shape[1]
    bm = 4160 if (M > 4160 and M % 4160 == 0) else M
    grid = M // bm
    return pl.pallas_call(
        functools.partial(_mm_body, relu=relu),
        grid=(grid,),
        in_specs=[
            pl.BlockSpec((bm, K), lambda i: (i, 0)),
            pl.BlockSpec((K, N), lambda i: (0, 0)),
        ],
        out_specs=pl.BlockSpec((bm, N), lambda i: (i, 0)),
        out_shape=jax.ShapeDtypeStruct((M, N), _F32),
    )(x, w)


def _deconv_body(x3_ref, we_ref, wo_ref, o_ref, *, C, O, relu):
    x3 = x3_ref[...]
    ev = jnp.dot(x3[:, : 2 * C], we_ref[...], preferred_element_type=_F32)
    od = jnp.dot(x3[:, C:], wo_ref[...], preferred_element_type=_F32)
    if relu:
        ev = jnp.maximum(ev, 0.0)
        od = jnp.maximum(od, 0.0)
    o_ref[:, :O] = ev
    o_ref[:, O:] = od


def _deconv_mm(x3, we, wo, relu):
    M, threeC = x3.shape
    C = threeC // 3
    O = we.shape[1]
    bm = 4160 if (M > 4160 and M % 4160 == 0) else M
    grid = M // bm
    return pl.pallas_call(
        functools.partial(_deconv_body, C=C, O=O, relu=relu),
        grid=(grid,),
        in_specs=[
            pl.BlockSpec((bm, threeC), lambda i: (i, 0)),
            pl.BlockSpec((2 * C, O), lambda i: (0, 0)),
            pl.BlockSpec((2 * C, O), lambda i: (0, 0)),
        ],
        out_specs=pl.BlockSpec((bm, 2 * O), lambda i: (i, 0)),
        out_shape=jax.ShapeDtypeStruct((M, 2 * O), _F32),
    )(x3, we, wo)


def _dist_body(z_ref, e_ref, idx_ref, perp_ref, acc_ref):
    k = pl.program_id(0)
    z = z_ref[...]
    e = e_ref[...]
    part = -2.0 * lax.dot_general(
        z, e, (((1,), (1,)), ((), ())), preferred_element_type=_F32)
    part = part + jnp.sum(e * e, axis=1, keepdims=True).T

    @pl.when(k == 0)
    def _():
        acc_ref[...] = part

    @pl.when(k > 0)
    def _():
        acc_ref[...] = acc_ref[...] + part

    @pl.when(k == pl.num_programs(0) - 1)
    def _():
        acc = acc_ref[...]
        minval = jnp.min(acc, axis=1, keepdims=True)
        colids = lax.broadcasted_iota(jnp.int32, acc.shape, 1)
        big = jnp.int32(2 ** 30)
        idx = jnp.min(jnp.where(acc == minval, colids, big),
                      axis=1, keepdims=True)
        idx_ref[...] = jnp.broadcast_to(idx, idx_ref.shape)
        onehot = (lax.broadcasted_iota(jnp.int32, acc.shape, 1)
                  == idx).astype(_F32)
        e_mean = jnp.sum(onehot, axis=0, keepdims=True) / acc.shape[0]
        perp = jnp.exp(-jnp.sum(e_mean * jnp.log(e_mean + 1e-10)))
        perp_ref[0, 0] = perp


def _vq_argmin(z_flat, E):
    B, D = z_flat.shape
    K = E.shape[0]
    KB = 1664
    grid = D // KB
    idx2d, perp = pl.pallas_call(
        _dist_body,
        grid=(grid,),
        in_specs=[
            pl.BlockSpec((B, KB), lambda i: (0, i)),
            pl.BlockSpec((K, KB), lambda i: (0, i)),
        ],
        out_specs=[
            pl.BlockSpec((B, 128), lambda i: (0, 0)),
            pl.BlockSpec((1, 1), lambda i: (0, 0), memory_space=pltpu.SMEM),
        ],
        out_shape=[
            jax.ShapeDtypeStruct((B, 128), jnp.int32),
            jax.ShapeDtypeStruct((1, 1), _F32),
        ],
        scratch_shapes=[pltpu.VMEM((B, K), _F32)],
    )(z_flat, E)
    return idx2d[:, 0], perp[0, 0]


def _sc_gather(E, idx):
    B = idx.shape[0]
    D = E.shape[1]
    rows_per_w = 8
    n_workers = B // rows_per_w
    mesh = plsc.VectorSubcoreMesh(core_axis_name="c", subcore_axis_name="s")

    @functools.partial(
        pl.kernel,
        mesh=mesh,
        out_type=jax.ShapeDtypeStruct((B, D), _F32),
        scratch_types=[
            pltpu.VMEM((rows_per_w,), jnp.int32),
            pltpu.VMEM((rows_per_w, D), _F32),
            pltpu.SemaphoreType.DMA,
        ],
    )
    def gather_kernel(idx_hbm, table_hbm, out_hbm, idx_v, rows_v, sem):
        wid = lax.axis_index("s") * 2 + lax.axis_index("c")

        @pl.when(wid < n_workers)
        def _():
            base = wid * rows_per_w
            pltpu.sync_copy(idx_hbm.at[pl.ds(base, rows_per_w)], idx_v)
            pltpu.async_copy(table_hbm.at[idx_v], rows_v, sem).wait()
            pltpu.sync_copy(rows_v, out_hbm.at[pl.ds(base, rows_per_w)])

    return gather_kernel(idx, E)


def _loss_body(z_ref, q_ref, o_ref):
    d = z_ref[...] - q_ref[...]
    o_ref[0, 0] = jnp.sum(d * d)


def _embedding_loss(z_flat, zq_flat):
    B, D = z_flat.shape
    s = pl.pallas_call(
        _loss_body,
        in_specs=[pl.BlockSpec((B, D), lambda: (0, 0)),
                  pl.BlockSpec((B, D), lambda: (0, 0))],
        out_specs=pl.BlockSpec((1, 1), lambda: (0, 0),
                               memory_space=pltpu.SMEM),
        out_shape=jax.ShapeDtypeStruct((1, 1), _F32),
    )(z_flat, zq_flat)
    return s[0, 0] * (1.25 / (B * D))


def _conv_im2col(x_tc):
    B, T_in, C = x_tc.shape
    T_out = T_in // 2
    xp = jnp.pad(x_tc, ((0, 0), (1, 1), (0, 0)))
    view = xp.reshape(B, T_out + 1, 2 * C)
    return jnp.concatenate([view[:, :T_out], view[:, 1:]], axis=-1)


def _deconv_cols(x_tc):
    B, T, C = x_tc.shape
    xp = jnp.pad(x_tc, ((0, 0), (1, 1), (0, 0)))
    return jnp.concatenate(
        [xp[:, :T], xp[:, 1:T + 1], xp[:, 2:T + 2]], axis=-1)


def _enc_w(w):
    return w.transpose(2, 1, 0).reshape(-1, w.shape[0])


def _dec_w(w):
    wt = w.transpose(2, 1, 0)
    O = w.shape[0]
    we = jnp.concatenate([wt[0], wt[2]], axis=0)
    wo = jnp.concatenate([wt[1], wt[3]], axis=0)
    return we, wo


def _head_w(w, nc_pad):
    N = w.shape[1]
    wr = w.reshape(64, 65, N).transpose(1, 0, 2).reshape(4160, N)
    if nc_pad > N:
        wr = jnp.pad(wr, ((0, 0), (0, nc_pad - N)))
    return wr


def kernel(x, We1, We2, We3, We4, E, Wm, bm, Wa1, ba1, Wa2, ba2, Wa3, ba3,
           Wd1, Wd2, Wd3, Wd4):
    B, T = x.shape

    h = x[:, :, None]
    for w, relu in ((We1, True), (We2, True), (We3, True), (We4, False)):
        xc = _conv_im2col(h)
        Bq, To, Kc = xc.shape
        o = _mm(xc.reshape(Bq * To, Kc), _enc_w(w), relu=relu)
        h = o.reshape(Bq, To, w.shape[0])
    z_e = h

    z_flat = z_e.transpose(0, 2, 1).reshape(B, -1)
    idx, perplexity = _vq_argmin(z_flat, E)
    zq_flat = _sc_gather(E, idx)
    embedding_loss = _embedding_loss(z_flat, zq_flat)

    zt_m = z_e[:, :, :64].reshape(B, -1)
    zt_a = z_e[:, :, 64:].reshape(B, -1)
    multitask = _mm(zt_m, _head_w(Wm, 128))[:, :Wm.shape[1]] + bm
    a = jnp.maximum(_mm(zt_a, _head_w(Wa1, Wa1.shape[1])) + ba1, 0.0)
    a = jnp.maximum(_mm(a, Wa2) + ba2, 0.0)
    adversary = _mm(a, jnp.pad(Wa3, ((0, 0), (0, 118))))[:, :Wa3.shape[1]] + ba3

    g = zq_flat.reshape(B, 128, 65).transpose(0, 2, 1)
    for w, relu in ((Wd1, True), (Wd2, True), (Wd3, True)):
        x3 = _deconv_cols(g)
        Bq, To, Kc = x3.shape
        we, wo = _dec_w(w)
        o = _deconv_mm(x3.reshape(Bq * To, Kc), we, wo, relu)
        g = o.reshape(Bq, To, 2, w.shape[0]).reshape(Bq, 2 * To, w.shape[0])
    x3 = _deconv_cols(g)
    Bq, To, Kc = x3.shape
    we4, wo4 = _dec_w(Wd4)
    C4 = Wd4.shape[1]
    wblk = jnp.concatenate([
        jnp.concatenate([we4[:C4], jnp.zeros((C4, 1), _F32)], axis=1),
        jnp.concatenate([we4[C4:], wo4[:C4]], axis=1),
        jnp.concatenate([jnp.zeros((C4, 1), _F32), wo4[C4:]], axis=1),
    ], axis=0)
    o = _mm(x3.reshape(Bq * To, Kc), wblk)
    x_hat = o.reshape(B, 2 * To)[:, None, :]

    return (embedding_loss, x_hat, multitask, adversary, perplexity)


# speedup vs baseline: 3.4749x; 3.4749x over previous
import functools

import jax
import jax.numpy as jnp
from jax import lax
from jax.experimental import pallas as pl
from jax.experimental.pallas import tpu as pltpu
from jax.experimental.pallas import tpu_sc as plsc

_F32 = jnp.float32


def _shift_down(p, zrow):
    return jnp.concatenate([zrow, p[:-1]], axis=0)


def _shift_up(p, zrow):
    return jnp.concatenate([p[1:], zrow], axis=0)


_NB = 8


def _enc_layer(pls, wcat, relu):
    U, C = pls[0][0].shape
    zrow = jnp.zeros((1, C), _F32)
    blocks = []
    for planes in pls:
        M = len(planes)
        for r in range(M // 2):
            a0 = (planes[2 * r - 1] if r > 0
                  else _shift_down(planes[M - 1], zrow))
            a1 = planes[2 * r]
            a2 = planes[2 * r + 1]
            a3 = (planes[2 * r + 2] if 2 * r + 2 < M
                  else _shift_up(planes[0], zrow))
            blocks.append(jnp.concatenate([a0, a1, a2, a3], axis=1))
    X = jnp.concatenate(blocks, axis=0)
    O = jnp.dot(X, wcat, preferred_element_type=_F32)
    if relu:
        O = jnp.maximum(O, 0.0)
    out, off = [], 0
    for planes in pls:
        half = len(planes) // 2
        out.append([O[(off + i) * U:(off + i + 1) * U] for i in range(half)])
        off += half
    return out


def _enc_body(x1_ref, w1_ref, w2_ref, w3_ref, w4_ref,
              zt_ref, zm_ref, za_ref):
    X = jnp.concatenate([x1_ref[b] for b in range(_NB)], axis=0)
    h = jnp.dot(X, w1_ref[...], preferred_element_type=_F32)
    h = jnp.maximum(h, 0.0)
    pls = [[h[(b * 8 + r) * 65:(b * 8 + r + 1) * 65] for r in range(8)]
           for b in range(_NB)]
    pls = _enc_layer(pls, w2_ref[...], True)
    pls = _enc_layer(pls, w3_ref[...], True)
    pls = _enc_layer(pls, w4_ref[...], False)
    for b in range(_NB):
        z = pls[b][0]
        zt_ref[b] = z.T
        zm_ref[b] = z[:, :64]
        za_ref[b] = z[:, 64:]


def _encoder(x1p, w1, w2, w3, w4):
    B = x1p.shape[0]
    return pl.pallas_call(
        _enc_body,
        grid=(B // _NB,),
        in_specs=[
            pl.BlockSpec((_NB, 520, 4), lambda b: (b, 0, 0)),
            pl.BlockSpec((4, 32), lambda b: (0, 0)),
            pl.BlockSpec((128, 64), lambda b: (0, 0)),
            pl.BlockSpec((256, 128), lambda b: (0, 0)),
            pl.BlockSpec((512, 128), lambda b: (0, 0)),
        ],
        out_specs=[
            pl.BlockSpec((_NB, 128, 65), lambda b: (b, 0, 0)),
            pl.BlockSpec((_NB, 65, 64), lambda b: (b, 0, 0)),
            pl.BlockSpec((_NB, 65, 64), lambda b: (b, 0, 0)),
        ],
        out_shape=[
            jax.ShapeDtypeStruct((B, 128, 65), _F32),
            jax.ShapeDtypeStruct((B, 65, 64), _F32),
            jax.ShapeDtypeStruct((B, 65, 64), _F32),
        ],
    )(x1p, w1, w2, w3, w4)


def _dec_shifted(planes):
    U, C = planes[0].shape
    M = len(planes)
    zrow = jnp.zeros((1, C), _F32)
    prev = [planes[r - 1] if r > 0 else _shift_down(planes[M - 1], zrow)
            for r in range(M)]
    nxt = [planes[r + 1] if r < M - 1 else _shift_up(planes[0], zrow)
           for r in range(M)]
    xe = [jnp.concatenate([prev[r], planes[r]], axis=1) for r in range(M)]
    xo = [jnp.concatenate([planes[r], nxt[r]], axis=1) for r in range(M)]
    return xe, xo


def _dec_layer(pls, we, wo, relu):
    U = pls[0][0].shape[0]
    xes, xos = [], []
    for planes in pls:
        xe, xo = _dec_shifted(planes)
        xes += xe
        xos += xo
    Ev = jnp.dot(jnp.concatenate(xes, 0), we, preferred_element_type=_F32)
    Od = jnp.dot(jnp.concatenate(xos, 0), wo, preferred_element_type=_F32)
    if relu:
        Ev = jnp.maximum(Ev, 0.0)
        Od = jnp.maximum(Od, 0.0)
    out, off = [], 0
    for planes in pls:
        M = len(planes)
        cur = []
        for r in range(M):
            cur.append(Ev[(off + r) * U:(off + r + 1) * U])
            cur.append(Od[(off + r) * U:(off + r + 1) * U])
        out.append(cur)
        off += M
    return out


def _dec_body(zq_ref, w1e_ref, w1o_ref, w2e_ref, w2o_ref,
              w3e_ref, w3o_ref, w4_ref, o_ref):
    pls = [[zq_ref[b].T] for b in range(_NB)]
    pls = _dec_layer(pls, w1e_ref[...], w1o_ref[...], True)
    pls = _dec_layer(pls, w2e_ref[...], w2o_ref[...], True)
    pls = _dec_layer(pls, w3e_ref[...], w3o_ref[...], True)
    xcat = []
    for planes in pls:
        xe, xo = _dec_shifted(planes)
        xcat += [jnp.concatenate([xe[r], xo[r]], axis=1) for r in range(8)]
    X4 = jnp.concatenate(xcat, 0)
    R = jnp.dot(X4, w4_ref[...], preferred_element_type=_F32)
    for b in range(_NB):
        for r in range(8):
            i = b * 8 + r
            o_ref[b, r] = R[i * 65:(i + 1) * 65]


def _decoder(zq3, w1e, w1o, w2e, w2o, w3e, w3o, w4):
    B = zq3.shape[0]
    return pl.pallas_call(
        _dec_body,
        grid=(B // _NB,),
        in_specs=[
            pl.BlockSpec((_NB, 128, 65), lambda b: (b, 0, 0)),
            pl.BlockSpec((256, 128), lambda b: (0, 0)),
            pl.BlockSpec((256, 128), lambda b: (0, 0)),
            pl.BlockSpec((256, 64), lambda b: (0, 0)),
            pl.BlockSpec((256, 64), lambda b: (0, 0)),
            pl.BlockSpec((128, 32), lambda b: (0, 0)),
            pl.BlockSpec((128, 32), lambda b: (0, 0)),
            pl.BlockSpec((128, 2), lambda b: (0, 0)),
        ],
        out_specs=pl.BlockSpec((_NB, 8, 65, 2), lambda b: (b, 0, 0, 0)),
        out_shape=jax.ShapeDtypeStruct((B, 8, 65, 2), _F32),
    )(zq3, w1e, w1o, w2e, w2o, w3e, w3o, w4)


def _dist_body(z_ref, e_ref, idx_ref, perp_ref, acc_ref):
    k = pl.program_id(0)
    z = z_ref[...]
    e = e_ref[...]
    part = -2.0 * lax.dot_general(
        z, e, (((1,), (1,)), ((), ())), preferred_element_type=_F32)
    part = part + jnp.sum(e * e, axis=1, keepdims=True).T

    @pl.when(k == 0)
    def _():
        acc_ref[...] = part

    @pl.when(k > 0)
    def _():
        acc_ref[...] = acc_ref[...] + part

    @pl.when(k == pl.num_programs(0) - 1)
    def _():
        acc = acc_ref[...]
        minval = jnp.min(acc, axis=1, keepdims=True)
        colids = lax.broadcasted_iota(jnp.int32, acc.shape, 1)
        big = jnp.int32(2 ** 30)
        idx = jnp.min(jnp.where(acc == minval, colids, big),
                      axis=1, keepdims=True)
        idx_ref[...] = jnp.broadcast_to(idx, idx_ref.shape)
        onehot = (colids == idx).astype(_F32)
        e_mean = jnp.sum(onehot, axis=0, keepdims=True) / acc.shape[0]
        perp = jnp.exp(-jnp.sum(e_mean * jnp.log(e_mean + 1e-10)))
        perp_ref[0, 0] = perp


def _vq_argmin(z_flat, E):
    B, D = z_flat.shape
    K = E.shape[0]
    KB = 1664
    grid = D // KB
    idx2d, perp = pl.pallas_call(
        _dist_body,
        grid=(grid,),
        in_specs=[
            pl.BlockSpec((B, KB), lambda i: (0, i)),
            pl.BlockSpec((K, KB), lambda i: (0, i)),
        ],
        out_specs=[
            pl.BlockSpec((B, 128), lambda i: (0, 0)),
            pl.BlockSpec((1, 1), lambda i: (0, 0), memory_space=pltpu.SMEM),
        ],
        out_shape=[
            jax.ShapeDtypeStruct((B, 128), jnp.int32),
            jax.ShapeDtypeStruct((1, 1), _F32),
        ],
        scratch_shapes=[pltpu.VMEM((B, K), _F32)],
    )(z_flat, E)
    return idx2d[:, 0], perp[0, 0]


def _sc_gather(E, idx):
    B = idx.shape[0]
    D = E.shape[1]
    rows_per_w = 8
    n_workers = B // rows_per_w
    mesh = plsc.VectorSubcoreMesh(core_axis_name="c", subcore_axis_name="s")

    @functools.partial(
        pl.kernel,
        mesh=mesh,
        out_type=jax.ShapeDtypeStruct((B, D), _F32),
        scratch_types=[
            pltpu.VMEM((rows_per_w,), jnp.int32),
            pltpu.VMEM((rows_per_w, D), _F32),
            pltpu.SemaphoreType.DMA,
        ],
    )
    def gather_kernel(idx_hbm, table_hbm, out_hbm, idx_v, rows_v, sem):
        wid = lax.axis_index("s") * 2 + lax.axis_index("c")

        @pl.when(wid < n_workers)
        def _():
            base = wid * rows_per_w
            pltpu.sync_copy(idx_hbm.at[pl.ds(base, rows_per_w)], idx_v)
            pltpu.async_copy(table_hbm.at[idx_v], rows_v, sem).wait()
            pltpu.sync_copy(rows_v, out_hbm.at[pl.ds(base, rows_per_w)])

    return gather_kernel(idx, E)


def _heads_body(zm_ref, za_ref, wm_ref, bm_ref, wa1_ref, ba1_ref,
                wa2_ref, ba2_ref, wa3_ref, ba3_ref, mt_ref, adv_ref):
    mt_ref[...] = (jnp.dot(zm_ref[...], wm_ref[...],
                           preferred_element_type=_F32) + bm_ref[...])
    a = jnp.dot(za_ref[...], wa1_ref[...], preferred_element_type=_F32)
    a = jnp.maximum(a + ba1_ref[...], 0.0)
    a = jnp.dot(a, wa2_ref[...], preferred_element_type=_F32)
    a = jnp.maximum(a + ba2_ref[...], 0.0)
    a = jnp.dot(a, wa3_ref[...], preferred_element_type=_F32)
    adv_ref[...] = a + ba3_ref[...]


def _heads(zm, za, wm, bm2, wa1, ba12, wa2, ba22, wa3, ba32):
    B = zm.shape[0]
    NC = wm.shape[1]
    full = lambda a: pl.BlockSpec(a.shape, lambda: tuple([0] * a.ndim))
    args = (zm, za, wm, bm2, wa1, ba12, wa2, ba22, wa3, ba32)
    return pl.pallas_call(
        _heads_body,
        in_specs=[full(a) for a in args],
        out_specs=[pl.BlockSpec((B, NC), lambda: (0, 0))] * 2,
        out_shape=[jax.ShapeDtypeStruct((B, NC), _F32)] * 2,
    )(*args)


def _loss_body(z_ref, q_ref, o_ref):
    d = z_ref[...] - q_ref[...]
    o_ref[0, 0] = jnp.sum(d * d)


def _embedding_loss(z_flat, zq_flat):
    B, D = z_flat.shape
    s = pl.pallas_call(
        _loss_body,
        in_specs=[pl.BlockSpec((B, D), lambda: (0, 0)),
                  pl.BlockSpec((B, D), lambda: (0, 0))],
        out_specs=pl.BlockSpec((1, 1), lambda: (0, 0),
                               memory_space=pltpu.SMEM),
        out_shape=jax.ShapeDtypeStruct((1, 1), _F32),
    )(z_flat, zq_flat)
    return s[0, 0] * (1.25 / (B * D))


def _enc_w(w):
    return w.transpose(2, 1, 0).reshape(-1, w.shape[0])


def _dec_w(w):
    wt = w.transpose(2, 1, 0)
    we = jnp.concatenate([wt[0], wt[2]], axis=0)
    wo = jnp.concatenate([wt[1], wt[3]], axis=0)
    return we, wo


def _head_w(w):
    N = w.shape[1]
    return w.reshape(64, 65, N).transpose(1, 0, 2).reshape(4160, N)


def kernel(x, We1, We2, We3, We4, E, Wm, bm, Wa1, ba1, Wa2, ba2, Wa3, ba3,
           Wd1, Wd2, Wd3, Wd4):
    B, T = x.shape

    xp = jnp.pad(x, ((0, 0), (1, 1)))
    view = xp.reshape(B, 521, 2)
    x1 = jnp.concatenate([view[:, :520], view[:, 1:]], axis=-1)
    x1p = x1.reshape(B, 65, 8, 4).transpose(0, 2, 1, 3).reshape(B, 520, 4)

    zT, zm3, za3 = _encoder(x1p, _enc_w(We1), _enc_w(We2),
                            _enc_w(We3), _enc_w(We4))
    z_flat = zT.reshape(B, -1)

    idx, perplexity = _vq_argmin(z_flat, E)
    zq_flat = _sc_gather(E, idx)
    embedding_loss = _embedding_loss(z_flat, zq_flat)

    r2 = lambda v: v.reshape(1, -1)
    multitask, adversary = _heads(
        zm3.reshape(B, 4160), za3.reshape(B, 4160),
        _head_w(Wm), r2(bm), _head_w(Wa1), r2(ba1),
        Wa2, r2(ba2), Wa3, r2(ba3))

    w1e, w1o = _dec_w(Wd1)
    w2e, w2o = _dec_w(Wd2)
    w3e, w3o = _dec_w(Wd3)
    wt4 = Wd4.transpose(2, 1, 0)
    z32 = jnp.zeros((64, 1), _F32)
    w4 = jnp.concatenate([
        jnp.concatenate([jnp.concatenate([wt4[0], wt4[2]], 0), z32], 1),
        jnp.concatenate([z32, jnp.concatenate([wt4[1], wt4[3]], 0)], 1),
    ], axis=0)

    xh = _decoder(zq_flat.reshape(B, 128, 65),
                  w1e, w1o, w2e, w2o, w3e, w3o, w4)
    x_hat = xh.transpose(0, 2, 1, 3).reshape(B, 1, 1040)

    return (embedding_loss, x_hat, multitask, adversary, perplexity)


# speedup vs baseline: 3.9275x; 1.1302x over previous
import functools

import jax
import jax.numpy as jnp
from jax import lax
from jax.experimental import pallas as pl
from jax.experimental.pallas import tpu as pltpu
from jax.experimental.pallas import tpu_sc as plsc

_F32 = jnp.float32


def _shift_down(p, zrow):
    return jnp.concatenate([zrow, p[:-1]], axis=0)


def _shift_up(p, zrow):
    return jnp.concatenate([p[1:], zrow], axis=0)


_NB = 8


def _enc_layer(pls, wcat, relu):
    U, C = pls[0][0].shape
    zrow = jnp.zeros((1, C), _F32)
    blocks = []
    for planes in pls:
        M = len(planes)
        for r in range(M // 2):
            a0 = (planes[2 * r - 1] if r > 0
                  else _shift_down(planes[M - 1], zrow))
            a1 = planes[2 * r]
            a2 = planes[2 * r + 1]
            a3 = (planes[2 * r + 2] if 2 * r + 2 < M
                  else _shift_up(planes[0], zrow))
            blocks.append(jnp.concatenate([a0, a1, a2, a3], axis=1))
    X = jnp.concatenate(blocks, axis=0)
    O = jnp.dot(X, wcat, preferred_element_type=_F32)
    if relu:
        O = jnp.maximum(O, 0.0)
    out, off = [], 0
    for planes in pls:
        half = len(planes) // 2
        out.append([O[(off + i) * U:(off + i + 1) * U] for i in range(half)])
        off += half
    return out


def _enc_body(xv_ref, bigw_ref, wlo_ref, whi_ref, w2_ref, w3_ref, w4_ref,
              zt_ref):
    V = jnp.concatenate([xv_ref[b] for b in range(_NB)], axis=0)
    big = jnp.dot(V, bigw_ref[...], preferred_element_type=_F32)
    zrow1 = jnp.zeros((1, 1), _F32)
    pls = []
    for b in range(_NB):
        v = V[b * 65:(b + 1) * 65]
        g = big[b * 65:(b + 1) * 65]
        corr0 = _shift_down(v[:, 15:16], zrow1) * wlo_ref[...]
        corr7 = _shift_up(v[:, 0:1], zrow1) * whi_ref[...]
        planes = [g[:, 32 * r:32 * (r + 1)] for r in range(8)]
        planes[0] = planes[0] + corr0
        planes[7] = planes[7] + corr7
        pls.append([jnp.maximum(p, 0.0) for p in planes])
    pls = _enc_layer(pls, w2_ref[...], True)
    pls = _enc_layer(pls, w3_ref[...], True)
    pls = _enc_layer(pls, w4_ref[...], False)
    for b in range(_NB):
        zt_ref[b] = pls[b][0].T


def _encoder(xv, bigw, wlo, whi, w2, w3, w4):
    B = xv.shape[0]
    return pl.pallas_call(
        _enc_body,
        grid=(B // _NB,),
        in_specs=[
            pl.BlockSpec((_NB, 65, 16), lambda b: (b, 0, 0)),
            pl.BlockSpec((16, 256), lambda b: (0, 0)),
            pl.BlockSpec((1, 32), lambda b: (0, 0)),
            pl.BlockSpec((1, 32), lambda b: (0, 0)),
            pl.BlockSpec((128, 64), lambda b: (0, 0)),
            pl.BlockSpec((256, 128), lambda b: (0, 0)),
            pl.BlockSpec((512, 128), lambda b: (0, 0)),
        ],
        out_specs=pl.BlockSpec((_NB, 128, 65), lambda b: (b, 0, 0)),
        out_shape=jax.ShapeDtypeStruct((B, 128, 65), _F32),
    )(xv, bigw, wlo, whi, w2, w3, w4)


def _dec_shifted(planes):
    U, C = planes[0].shape
    M = len(planes)
    zrow = jnp.zeros((1, C), _F32)
    prev = [planes[r - 1] if r > 0 else _shift_down(planes[M - 1], zrow)
            for r in range(M)]
    nxt = [planes[r + 1] if r < M - 1 else _shift_up(planes[0], zrow)
           for r in range(M)]
    xe = [jnp.concatenate([prev[r], planes[r]], axis=1) for r in range(M)]
    xo = [jnp.concatenate([planes[r], nxt[r]], axis=1) for r in range(M)]
    return xe, xo


def _dec_layer(pls, we, wo, relu):
    U = pls[0][0].shape[0]
    xes, xos = [], []
    for planes in pls:
        xe, xo = _dec_shifted(planes)
        xes += xe
        xos += xo
    Ev = jnp.dot(jnp.concatenate(xes, 0), we, preferred_element_type=_F32)
    Od = jnp.dot(jnp.concatenate(xos, 0), wo, preferred_element_type=_F32)
    if relu:
        Ev = jnp.maximum(Ev, 0.0)
        Od = jnp.maximum(Od, 0.0)
    out, off = [], 0
    for planes in pls:
        M = len(planes)
        cur = []
        for r in range(M):
            cur.append(Ev[(off + r) * U:(off + r + 1) * U])
            cur.append(Od[(off + r) * U:(off + r + 1) * U])
        out.append(cur)
        off += M
    return out


def _dec_body(zq_ref, w1e_ref, w1o_ref, w2e_ref, w2o_ref,
              w3e_ref, w3o_ref, w4_ref, o_ref):
    pls = [[zq_ref[b].T] for b in range(_NB)]
    pls = _dec_layer(pls, w1e_ref[...], w1o_ref[...], True)
    pls = _dec_layer(pls, w2e_ref[...], w2o_ref[...], True)
    pls = _dec_layer(pls, w3e_ref[...], w3o_ref[...], True)
    xcat = []
    for planes in pls:
        xe, xo = _dec_shifted(planes)
        xcat += [jnp.concatenate([xe[r], xo[r]], axis=1) for r in range(8)]
    X4 = jnp.concatenate(xcat, 0)
    R = jnp.dot(X4, w4_ref[...], preferred_element_type=_F32)
    for b in range(_NB):
        o_ref[b] = jnp.concatenate(
            [R[(b * 8 + r) * 65:(b * 8 + r + 1) * 65] for r in range(8)],
            axis=1)


def _decoder(zq3, w1e, w1o, w2e, w2o, w3e, w3o, w4):
    B = zq3.shape[0]
    return pl.pallas_call(
        _dec_body,
        grid=(B // _NB,),
        in_specs=[
            pl.BlockSpec((_NB, 128, 65), lambda b: (b, 0, 0)),
            pl.BlockSpec((256, 128), lambda b: (0, 0)),
            pl.BlockSpec((256, 128), lambda b: (0, 0)),
            pl.BlockSpec((256, 64), lambda b: (0, 0)),
            pl.BlockSpec((256, 64), lambda b: (0, 0)),
            pl.BlockSpec((128, 32), lambda b: (0, 0)),
            pl.BlockSpec((128, 32), lambda b: (0, 0)),
            pl.BlockSpec((128, 2), lambda b: (0, 0)),
        ],
        out_specs=pl.BlockSpec((_NB, 65, 16), lambda b: (b, 0, 0)),
        out_shape=jax.ShapeDtypeStruct((B, 65, 16), _F32),
    )(zq3, w1e, w1o, w2e, w2o, w3e, w3o, w4)


def _dist_body(z_ref, e_ref, idx_ref, perp_ref, acc_ref, z2d_ref):
    k = pl.program_id(0)

    @pl.when(k == 0)
    def _():
        z2d_ref[...] = z_ref[...].reshape(z2d_ref.shape)

    KB = e_ref.shape[1]
    z = z2d_ref[:, pl.ds(k * KB, KB)]
    e = e_ref[...]
    part = -2.0 * lax.dot_general(
        z, e, (((1,), (1,)), ((), ())), preferred_element_type=_F32)
    part = part + jnp.sum(e * e, axis=1, keepdims=True).T

    @pl.when(k == 0)
    def _():
        acc_ref[...] = part

    @pl.when(k > 0)
    def _():
        acc_ref[...] = acc_ref[...] + part

    @pl.when(k == pl.num_programs(0) - 1)
    def _():
        acc = acc_ref[...]
        minval = jnp.min(acc, axis=1, keepdims=True)
        colids = lax.broadcasted_iota(jnp.int32, acc.shape, 1)
        big = jnp.int32(2 ** 30)
        idx = jnp.min(jnp.where(acc == minval, colids, big),
                      axis=1, keepdims=True)
        idx_ref[...] = jnp.broadcast_to(idx, idx_ref.shape)
        onehot = (colids == idx).astype(_F32)
        e_mean = jnp.sum(onehot, axis=0, keepdims=True) / acc.shape[0]
        perp = jnp.exp(-jnp.sum(e_mean * jnp.log(e_mean + 1e-10)))
        perp_ref[0, 0] = perp


def _vq_argmin(zT, E):
    B = zT.shape[0]
    K, D = E.shape
    KB = 1664
    grid = D // KB
    idx2d, perp = pl.pallas_call(
        _dist_body,
        grid=(grid,),
        in_specs=[
            pl.BlockSpec((B, 128, 65), lambda i: (0, 0, 0)),
            pl.BlockSpec((K, KB), lambda i: (0, i)),
        ],
        out_specs=[
            pl.BlockSpec((B, 128), lambda i: (0, 0)),
            pl.BlockSpec((1, 1), lambda i: (0, 0), memory_space=pltpu.SMEM),
        ],
        out_shape=[
            jax.ShapeDtypeStruct((B, 128), jnp.int32),
            jax.ShapeDtypeStruct((1, 1), _F32),
        ],
        scratch_shapes=[pltpu.VMEM((B, K), _F32), pltpu.VMEM((B, D), _F32)],
    )(zT, E)
    return idx2d[:, 0], perp[0, 0]


def _sc_gather(E, idx):
    B = idx.shape[0]
    D = E.shape[1]
    rows_per_w = 8
    n_workers = B // rows_per_w
    mesh = plsc.VectorSubcoreMesh(core_axis_name="c", subcore_axis_name="s")

    @functools.partial(
        pl.kernel,
        mesh=mesh,
        out_type=jax.ShapeDtypeStruct((B, D), _F32),
        scratch_types=[
            pltpu.VMEM((rows_per_w,), jnp.int32),
            pltpu.VMEM((rows_per_w, D), _F32),
            pltpu.SemaphoreType.DMA,
        ],
    )
    def gather_kernel(idx_hbm, table_hbm, out_hbm, idx_v, rows_v, sem):
        wid = lax.axis_index("s") * 2 + lax.axis_index("c")

        @pl.when(wid < n_workers)
        def _():
            base = wid * rows_per_w
            pltpu.sync_copy(idx_hbm.at[pl.ds(base, rows_per_w)], idx_v)
            pltpu.async_copy(table_hbm.at[idx_v], rows_v, sem).wait()
            pltpu.sync_copy(rows_v, out_hbm.at[pl.ds(base, rows_per_w)])

    return gather_kernel(idx, E)


def _heads_body(zt_ref, wm_ref, bm_ref, wa1_ref, ba1_ref,
                wa2_ref, ba2_ref, wa3_ref, ba3_ref, mt_ref, adv_ref):
    zz = zt_ref[...]
    B = zz.shape[0]
    zm = zz[:, :64, :].reshape(B, 4160)
    za = zz[:, 64:, :].reshape(B, 4160)
    mt_ref[...] = (jnp.dot(zm, wm_ref[...],
                           preferred_element_type=_F32) + bm_ref[...])
    a = jnp.dot(za, wa1_ref[...], preferred_element_type=_F32)
    a = jnp.maximum(a + ba1_ref[...], 0.0)
    a = jnp.dot(a, wa2_ref[...], preferred_element_type=_F32)
    a = jnp.maximum(a + ba2_ref[...], 0.0)
    a = jnp.dot(a, wa3_ref[...], preferred_element_type=_F32)
    adv_ref[...] = a + ba3_ref[...]


def _heads(zT, wm, bm2, wa1, ba12, wa2, ba22, wa3, ba32):
    B = zT.shape[0]
    NC = wm.shape[1]
    full = lambda a: pl.BlockSpec(a.shape, lambda: tuple([0] * a.ndim))
    args = (zT, wm, bm2, wa1, ba12, wa2, ba22, wa3, ba32)
    return pl.pallas_call(
        _heads_body,
        in_specs=[full(a) for a in args],
        out_specs=[pl.BlockSpec((B, NC), lambda: (0, 0))] * 2,
        out_shape=[jax.ShapeDtypeStruct((B, NC), _F32)] * 2,
    )(*args)


def _loss_body(z_ref, q_ref, o_ref):
    d = z_ref[...].reshape(q_ref.shape) - q_ref[...]
    o_ref[0, 0] = jnp.sum(d * d)


def _embedding_loss(zT, zq_flat):
    B, D = zq_flat.shape
    s = pl.pallas_call(
        _loss_body,
        in_specs=[pl.BlockSpec((B, 128, 65), lambda: (0, 0, 0)),
                  pl.BlockSpec((B, D), lambda: (0, 0))],
        out_specs=pl.BlockSpec((1, 1), lambda: (0, 0),
                               memory_space=pltpu.SMEM),
        out_shape=jax.ShapeDtypeStruct((1, 1), _F32),
    )(zT, zq_flat)
    return s[0, 0] * (1.25 / (B * D))


def _enc_w(w):
    return w.transpose(2, 1, 0).reshape(-1, w.shape[0])


def _dec_w(w):
    wt = w.transpose(2, 1, 0)
    we = jnp.concatenate([wt[0], wt[2]], axis=0)
    wo = jnp.concatenate([wt[1], wt[3]], axis=0)
    return we, wo


def kernel(x, We1, We2, We3, We4, E, Wm, bm, Wa1, ba1, Wa2, ba2, Wa3, ba3,
           Wd1, Wd2, Wd3, Wd4):
    B, T = x.shape

    xv = x.reshape(B, 65, 16)
    w1cat = _enc_w(We1)
    bigw = jnp.zeros((16, 8, 32), _F32)
    for r in range(8):
        for j in range(4):
            k = 2 * r - 1 + j
            if 0 <= k < 16:
                bigw = bigw.at[k, r].set(w1cat[j])
    bigw = bigw.reshape(16, 256)
    wlo = w1cat[0:1]
    whi = w1cat[3:4]

    zT = _encoder(xv, bigw, wlo, whi, _enc_w(We2),
                  _enc_w(We3), _enc_w(We4))

    idx, perplexity = _vq_argmin(zT, E)
    zq_flat = _sc_gather(E, idx)
    embedding_loss = _embedding_loss(zT, zq_flat)

    r2 = lambda v: v.reshape(1, -1)
    multitask, adversary = _heads(
        zT, Wm, r2(bm), Wa1, r2(ba1), Wa2, r2(ba2), Wa3, r2(ba3))

    w1e, w1o = _dec_w(Wd1)
    w2e, w2o = _dec_w(Wd2)
    w3e, w3o = _dec_w(Wd3)
    wt4 = Wd4.transpose(2, 1, 0)
    z32 = jnp.zeros((64, 1), _F32)
    w4 = jnp.concatenate([
        jnp.concatenate([jnp.concatenate([wt4[0], wt4[2]], 0), z32], 1),
        jnp.concatenate([z32, jnp.concatenate([wt4[1], wt4[3]], 0)], 1),
    ], axis=0)

    xh = _decoder(zq_flat.reshape(B, 128, 65),
                  w1e, w1o, w2e, w2o, w3e, w3o, w4)
    x_hat = xh.reshape(B, 1, 1040)

    return (embedding_loss, x_hat, multitask, adversary, perplexity)
